# baseline (speedup 1.0000x reference)
import functools

import jax
import jax.numpy as jnp
from jax import lax
from jax.experimental import pallas as pl
from jax.experimental.pallas import tpu as pltpu
from jax.experimental.pallas import tpu_sc as plsc

B, N, C = 4, 10000, 256
NH, NP = 4, 20
H = W = 100
DH = C // NH
KI = 2 * NP
KW = 4 * NP
ITEMS = B * NH * N
BN = 1000
CQ = 200


def _ph1_body(q_ref, qp_ref, wvp_ref, bvp_ref, wsx_ref, bsx_ref, wsy_ref,
              bsy_ref, waw_ref, baw_ref, v_ref, idx_ref, w_ref):
    b = pl.program_id(0)
    nb = pl.program_id(1)
    q = q_ref[0] + qp_ref[0]
    nvec = nb * BN + lax.broadcasted_iota(jnp.int32, (BN, 1), 0)
    row = (nvec // W).astype(jnp.float32)
    col = (nvec % W).astype(jnp.float32)
    dn = (((1,), (1,)), ((), ()))
    for h in range(NH):
        val_h = lax.dot_general(q, wvp_ref[h], dn) + bvp_ref[h]
        sox = lax.dot_general(q, wsx_ref[h], dn) + bsx_ref[h]
        soy = lax.dot_general(q, wsy_ref[h], dn) + bsy_ref[h]
        logits = lax.dot_general(q, waw_ref[h], dn) + baw_ref[h]
        m = jnp.max(logits, axis=-1, keepdims=True)
        e = jnp.exp(logits - m)
        aw = e / jnp.sum(e, axis=-1, keepdims=True)
        ix = row + sox
        iy = col + soy
        bx = jnp.clip(jnp.floor(ix), 0.0, W - 1.0)
        by = jnp.clip(jnp.floor(iy), 0.0, H - 1.0)
        wa = jnp.maximum(1.0 - jnp.abs(ix - bx), 0.0) * aw
        wb = (jnp.maximum(1.0 - jnp.abs(ix - bx - 1.0), 0.0) * aw
              * (bx <= W - 2.0))
        wy1 = jnp.maximum(1.0 - jnp.abs(iy - by), 0.0)
        wy2 = (jnp.maximum(1.0 - jnp.abs(iy - by - 1.0), 0.0)
               * (by <= H - 2.0))
        bxi = bx.astype(jnp.int32)
        byi = by.astype(jnp.int32)
        by2i = jnp.minimum(byi + 1, H - 1)
        base = (b * NH + h) * (H * W)
        r1 = jnp.clip(base + byi * W + bxi, 0, ITEMS - 1)
        r2 = jnp.clip(base + by2i * W + bxi, 0, ITEMS - 1)
        val_nxt = jnp.concatenate([val_h[1:], val_h[BN - 1:]], axis=0)
        v_ref[0, h] = jnp.concatenate([val_h, val_nxt], axis=1)
        idx_ref[0, h] = jnp.concatenate([r1, r2], axis=1)
        w_ref[0, h] = jnp.concatenate(
            [wy1 * wa, wy1 * wb, wy2 * wa, wy2 * wb], axis=1)


def _tr_in_body(i_ref, w_ref, it_ref, wt_ref):
    x = i_ref[0, 0]
    it_ref[0, 0] = x.reshape(W, W, KI).transpose(1, 0, 2).reshape(N, KI)
    y = w_ref[0, 0]
    wt_ref[0, 0] = y.reshape(W, W, KW).transpose(1, 0, 2).reshape(N, KW)


def _tr_out_body(x_ref, o_ref):
    x = x_ref[0, 0]
    o_ref[0, 0] = x.reshape(W, W, DH).transpose(1, 0, 2).reshape(N, DH)


def _ph3_body(x0_ref, x1_ref, x2_ref, x3_ref, q_ref, wop_ref, bop_ref,
              wout_ref, bout_ref, o_ref):
    dn = (((1,), (1,)), ((), ()))
    xs = (x0_ref, x1_ref, x2_ref, x3_ref)
    t = bop_ref[...]
    for h in range(NH):
        t = t + lax.dot_general(xs[h][...], wop_ref[h], dn)
    o_ref[...] = (lax.dot_general(t, wout_ref[...], dn) + bout_ref[...]
                  + q_ref[...])


def _sc_gather(v_flat, idx_flat, w_flat):
    info = plsc.get_sparse_core_info()
    nc, ns = info.num_cores, info.num_subcores
    nw = nc * ns
    ipt = ITEMS // nw
    nchunks = ipt // CQ

    mesh = plsc.VectorSubcoreMesh(core_axis_name="c", subcore_axis_name="s")

    gq = 1
    ng = CQ // gq
    idx4 = idx_flat.reshape(nw, nchunks, ng, gq * KI)

    @functools.partial(
        pl.kernel, mesh=mesh,
        out_type=jax.ShapeDtypeStruct((NH, B * N, DH), jnp.float32),
        scratch_types=[
            pltpu.VMEM((ng, gq * KI), jnp.int32),
            pltpu.VMEM((CQ, KW), jnp.float32),
            pltpu.VMEM((2, gq * KI, 2 * DH), jnp.float32),
            pltpu.VMEM((CQ, DH), jnp.float32),
            pltpu.SemaphoreType.DMA,
            pltpu.SemaphoreType.DMA,
        ],
    )
    def k(v_hbm, idx_hbm, w_hbm, out_hbm, idx_v, w_v, rows_v, out_v, sem0,
          sem1):
        wid = lax.axis_index("s") * nc + lax.axis_index("c")
        start = wid * ipt
        bh = start // N
        b = bh // NH
        h = bh % NH
        nbase = start % N

        def issue(gi, buf, sem):
            pltpu.async_copy(v_hbm.at[idx_v.at[gi]], rows_v.at[buf], sem)

        def gwait(gi, buf, sem):
            pltpu.make_async_copy(v_hbm.at[idx_v.at[gi]], rows_v.at[buf],
                                  sem).wait()

        def compute(gi, buf):
            for jg in range(gq):
                qi = gi * gq + jg
                wvec = [w_v[qi, pl.ds(k16 * 16, 16)] for k16 in range(5)]

                def bcast(lane, wvec=wvec):
                    return lax.gather(
                        wvec[lane // 16],
                        jnp.full((16, 1), lane % 16, jnp.int32),
                        lax.GatherDimensionNumbers(
                            offset_dims=(), collapsed_slice_dims=(0,),
                            start_index_map=(0,)),
                        (1,),
                        mode=lax.GatherScatterMode.PROMISE_IN_BOUNDS)

                acc = [jnp.zeros((16,), jnp.float32) for _ in range(4)]
                for s in range(KI):
                    la = s if s < NP else KI + (s - NP)
                    lb = NP + s if s < NP else KI + NP + (s - NP)
                    wva = bcast(la)
                    wvb = bcast(lb)
                    for t in range(4):
                        acc[t] = (acc[t]
                                  + wva * rows_v[buf, jg * KI + s,
                                                 pl.ds(t * 16, 16)]
                                  + wvb * rows_v[buf, jg * KI + s,
                                                 pl.ds(DH + t * 16, 16)])
                for t in range(4):
                    out_v[qi, pl.ds(t * 16, 16)] = acc[t]

        def chunk_body(ci, _):
            ibase = start + ci * CQ
            pltpu.sync_copy(idx_hbm.at[wid, ci], idx_v)
            pltpu.sync_copy(w_hbm.at[pl.ds(ibase, CQ)], w_v)
            issue(0, 0, sem0)

            def gg_body(gg, _):
                g0 = 2 * gg
                issue(g0 + 1, 1, sem1)
                gwait(g0, 0, sem0)
                compute(g0, 0)
                issue(jnp.minimum(g0 + 2, ng - 1), 0, sem0)
                gwait(g0 + 1, 1, sem1)
                compute(g0 + 1, 1)
                return 0

            lax.fori_loop(0, ng // 2, gg_body, 0)
            gwait(ng - 1, 0, sem0)
            pltpu.sync_copy(
                out_v,
                out_hbm.at[h, pl.ds(b * N + nbase + ci * CQ, CQ)])
            return 0

        lax.fori_loop(0, nchunks, chunk_body, 0)

    return k(v_flat, idx4, w_flat)


def kernel(query, query_pos, W_so, b_so, W_aw, b_aw, W_vp, b_vp, W_op, b_op,
           W_out, b_out):
    wvp = W_vp.reshape(NH, DH, C)
    bvp = b_vp.reshape(NH, DH)
    wso = W_so.reshape(NH, NP, 2, C)
    bso = b_so.reshape(NH, NP, 2)
    waw = W_aw.reshape(NH, NP, C)
    baw = b_aw.reshape(NH, NP)

    v, idx, w = pl.pallas_call(
        _ph1_body,
        grid=(B, N // BN),
        in_specs=[
            pl.BlockSpec((1, BN, C), lambda b, i: (b, i, 0)),
            pl.BlockSpec((1, BN, C), lambda b, i: (b, i, 0)),
            pl.BlockSpec((NH, DH, C), lambda b, i: (0, 0, 0)),
            pl.BlockSpec((NH, DH), lambda b, i: (0, 0)),
            pl.BlockSpec((NH, NP, C), lambda b, i: (0, 0, 0)),
            pl.BlockSpec((NH, NP), lambda b, i: (0, 0)),
            pl.BlockSpec((NH, NP, C), lambda b, i: (0, 0, 0)),
            pl.BlockSpec((NH, NP), lambda b, i: (0, 0)),
            pl.BlockSpec((NH, NP, C), lambda b, i: (0, 0, 0)),
            pl.BlockSpec((NH, NP), lambda b, i: (0, 0)),
        ],
        out_specs=[
            pl.BlockSpec((1, NH, BN, 2 * DH), lambda b, i: (b, 0, i, 0)),
            pl.BlockSpec((1, NH, BN, KI), lambda b, i: (b, 0, i, 0)),
            pl.BlockSpec((1, NH, BN, KW), lambda b, i: (b, 0, i, 0)),
        ],
        out_shape=[
            jax.ShapeDtypeStruct((B, NH, N, 2 * DH), jnp.float32),
            jax.ShapeDtypeStruct((B, NH, N, KI), jnp.int32),
            jax.ShapeDtypeStruct((B, NH, N, KW), jnp.float32),
        ],
        compiler_params=pltpu.CompilerParams(
            dimension_semantics=("parallel", "parallel")),
    )(query, query_pos, wvp, bvp, wso[:, :, 0], bso[:, :, 0], wso[:, :, 1],
      bso[:, :, 1], waw, baw)

    idx_t, w_t = pl.pallas_call(
        _tr_in_body,
        grid=(B, NH),
        in_specs=[
            pl.BlockSpec((1, 1, N, KI), lambda b, h: (b, h, 0, 0)),
            pl.BlockSpec((1, 1, N, KW), lambda b, h: (b, h, 0, 0)),
        ],
        out_specs=[
            pl.BlockSpec((1, 1, N, KI), lambda b, h: (b, h, 0, 0)),
            pl.BlockSpec((1, 1, N, KW), lambda b, h: (b, h, 0, 0)),
        ],
        out_shape=[
            jax.ShapeDtypeStruct((B, NH, N, KI), jnp.int32),
            jax.ShapeDtypeStruct((B, NH, N, KW), jnp.float32),
        ],
        compiler_params=pltpu.CompilerParams(
            dimension_semantics=("parallel", "parallel")),
    )(idx, w)

    out_t = _sc_gather(v.reshape(ITEMS, 2 * DH), idx_t.reshape(ITEMS, KI),
                       w_t.reshape(ITEMS, KW))

    out_s = pl.pallas_call(
        _tr_out_body,
        grid=(NH, B),
        in_specs=[pl.BlockSpec((1, 1, N, DH), lambda h, b: (h, b, 0, 0))],
        out_specs=pl.BlockSpec((1, 1, N, DH), lambda h, b: (h, b, 0, 0)),
        out_shape=jax.ShapeDtypeStruct((NH, B, N, DH), jnp.float32),
        compiler_params=pltpu.CompilerParams(
            dimension_semantics=("parallel", "parallel")),
    )(out_t.reshape(NH, B, N, DH)).reshape(NH, B * N, DH)

    qfl = query.reshape(B * N, C)
    wop_h = jnp.transpose(W_op.reshape(C, NH, DH), (1, 0, 2))
    BN3 = 1000
    out = pl.pallas_call(
        _ph3_body,
        grid=(B * N // BN3,),
        in_specs=[
            pl.BlockSpec((BN3, DH), lambda i: (i, 0)),
            pl.BlockSpec((BN3, DH), lambda i: (i, 0)),
            pl.BlockSpec((BN3, DH), lambda i: (i, 0)),
            pl.BlockSpec((BN3, DH), lambda i: (i, 0)),
            pl.BlockSpec((BN3, C), lambda i: (i, 0)),
            pl.BlockSpec((NH, C, DH), lambda i: (0, 0, 0)),
            pl.BlockSpec((C,), lambda i: (0,)),
            pl.BlockSpec((C, C), lambda i: (0, 0)),
            pl.BlockSpec((C,), lambda i: (0,)),
        ],
        out_specs=pl.BlockSpec((BN3, C), lambda i: (i, 0)),
        out_shape=jax.ShapeDtypeStruct((B * N, C), jnp.float32),
        compiler_params=pltpu.CompilerParams(
            dimension_semantics=("parallel",)),
    )(out_s[0], out_s[1], out_s[2], out_s[3], qfl, wop_h, b_op, W_out, b_out)
    return out.reshape(B, N, C)


# speedup vs baseline: 1.0947x; 1.0947x over previous
import functools

import jax
import jax.numpy as jnp
from jax import lax
from jax.experimental import pallas as pl
from jax.experimental.pallas import tpu as pltpu
from jax.experimental.pallas import tpu_sc as plsc

B, N, C = 4, 10000, 256
NH, NP = 4, 20
H = W = 100
DH = C // NH
KI = 2 * NP
KW = 4 * NP
ITEMS = B * NH * N
BN = 1000
CQ = 200


def _ph1_body(q_ref, qp_ref, wvp_ref, bvp_ref, wsx_ref, bsx_ref, wsy_ref,
              bsy_ref, waw_ref, baw_ref, v_ref, idx_ref, w_ref):
    b = pl.program_id(0)
    nb = pl.program_id(1)
    q = q_ref[0] + qp_ref[0]
    nvec = nb * BN + lax.broadcasted_iota(jnp.int32, (BN, 1), 0)
    row = (nvec // W).astype(jnp.float32)
    col = (nvec % W).astype(jnp.float32)
    dn = (((1,), (1,)), ((), ()))
    for h in range(NH):
        val_h = lax.dot_general(q, wvp_ref[h], dn) + bvp_ref[h]
        sox = lax.dot_general(q, wsx_ref[h], dn) + bsx_ref[h]
        soy = lax.dot_general(q, wsy_ref[h], dn) + bsy_ref[h]
        logits = lax.dot_general(q, waw_ref[h], dn) + baw_ref[h]
        m = jnp.max(logits, axis=-1, keepdims=True)
        e = jnp.exp(logits - m)
        aw = e / jnp.sum(e, axis=-1, keepdims=True)
        ix = row + sox
        iy = col + soy
        bx = jnp.clip(jnp.floor(ix), 0.0, W - 1.0)
        by = jnp.clip(jnp.floor(iy), 0.0, H - 1.0)
        wa = jnp.maximum(1.0 - jnp.abs(ix - bx), 0.0) * aw
        wb = (jnp.maximum(1.0 - jnp.abs(ix - bx - 1.0), 0.0) * aw
              * (bx <= W - 2.0))
        wy1 = jnp.maximum(1.0 - jnp.abs(iy - by), 0.0)
        wy2 = (jnp.maximum(1.0 - jnp.abs(iy - by - 1.0), 0.0)
               * (by <= H - 2.0))
        bxi = bx.astype(jnp.int32)
        byi = by.astype(jnp.int32)
        by2i = jnp.minimum(byi + 1, H - 1)
        base = (b * NH + h) * (H * W)
        r1 = jnp.clip(base + byi * W + bxi, 0, ITEMS - 1)
        r2 = jnp.clip(base + by2i * W + bxi, 0, ITEMS - 1)
        val_nxt = jnp.concatenate([val_h[1:], val_h[BN - 1:]], axis=0)
        v_ref[0, h] = jnp.concatenate([val_h, val_nxt], axis=1)
        idx_ref[0, h] = jnp.concatenate([r1, r2], axis=1)
        w_ref[0, h] = jnp.concatenate(
            [wy1 * wa, wy1 * wb, wy2 * wa, wy2 * wb], axis=1)


def _ph3_body(x0_ref, x1_ref, x2_ref, x3_ref, q_ref, wop_ref, bop_ref,
              wout_ref, bout_ref, o_ref):
    dn = (((1,), (1,)), ((), ()))
    xs = (x0_ref, x1_ref, x2_ref, x3_ref)
    t = bop_ref[...]
    for h in range(NH):
        t = t + lax.dot_general(xs[h][...], wop_ref[h], dn)
    o_ref[...] = (lax.dot_general(t, wout_ref[...], dn) + bout_ref[...]
                  + q_ref[...])


def _sc_gather(v_flat, idx_flat, w_flat):
    info = plsc.get_sparse_core_info()
    nc, ns = info.num_cores, info.num_subcores
    nw = nc * ns
    ipt = ITEMS // nw
    nchunks = ipt // CQ

    mesh = plsc.VectorSubcoreMesh(core_axis_name="c", subcore_axis_name="s")

    gq = 1
    ng = CQ // gq
    idx4 = idx_flat.reshape(nw, nchunks, ng, gq * KI)

    @functools.partial(
        pl.kernel, mesh=mesh,
        out_type=jax.ShapeDtypeStruct((NH, B * N, DH), jnp.float32),
        scratch_types=[
            pltpu.VMEM((ng, gq * KI), jnp.int32),
            pltpu.VMEM((CQ, KW), jnp.float32),
            pltpu.VMEM((4, gq * KI, 2 * DH), jnp.float32),
            pltpu.VMEM((CQ, DH), jnp.float32),
            pltpu.SemaphoreType.DMA,
            pltpu.SemaphoreType.DMA,
            pltpu.SemaphoreType.DMA,
            pltpu.SemaphoreType.DMA,
        ],
    )
    def k(v_hbm, idx_hbm, w_hbm, out_hbm, idx_v, w_v, rows_v, out_v, sem0,
          sem1, sem2, sem3):
        sems = (sem0, sem1, sem2, sem3)
        wid = lax.axis_index("s") * nc + lax.axis_index("c")
        start = wid * ipt
        bh = start // N
        b = bh // NH
        h = bh % NH
        nbase = start % N

        def issue(gi, buf, sem):
            pltpu.async_copy(v_hbm.at[idx_v.at[gi]], rows_v.at[buf], sem)

        def gwait(gi, buf, sem):
            pltpu.make_async_copy(v_hbm.at[idx_v.at[gi]], rows_v.at[buf],
                                  sem).wait()

        def compute(gi, buf):
            for jg in range(gq):
                qi = gi * gq + jg
                wvec = [w_v[qi, pl.ds(k16 * 16, 16)] for k16 in range(5)]

                def bcast(lane, wvec=wvec):
                    return lax.gather(
                        wvec[lane // 16],
                        jnp.full((16, 1), lane % 16, jnp.int32),
                        lax.GatherDimensionNumbers(
                            offset_dims=(), collapsed_slice_dims=(0,),
                            start_index_map=(0,)),
                        (1,),
                        mode=lax.GatherScatterMode.PROMISE_IN_BOUNDS)

                acc = [jnp.zeros((16,), jnp.float32) for _ in range(4)]
                for s in range(KI):
                    la = s if s < NP else KI + (s - NP)
                    lb = NP + s if s < NP else KI + NP + (s - NP)
                    wva = bcast(la)
                    wvb = bcast(lb)
                    for t in range(4):
                        acc[t] = (acc[t]
                                  + wva * rows_v[buf, jg * KI + s,
                                                 pl.ds(t * 16, 16)]
                                  + wvb * rows_v[buf, jg * KI + s,
                                                 pl.ds(DH + t * 16, 16)])
                for t in range(4):
                    out_v[qi, pl.ds(t * 16, 16)] = acc[t]

        def chunk_body(ci, _):
            ibase = start + ci * CQ
            pltpu.sync_copy(idx_hbm.at[wid, ci], idx_v)
            pltpu.sync_copy(w_hbm.at[pl.ds(ibase, CQ)], w_v)
            for p in range(3):
                issue(p, p, sems[p])

            def gg_body(gg, _):
                g0 = 4 * gg
                for p in range(4):
                    issue(jnp.minimum(g0 + p + 3, ng - 1), (p + 3) % 4,
                          sems[(p + 3) % 4])
                    gwait(g0 + p, p, sems[p])
                    compute(g0 + p, p)
                return 0

            lax.fori_loop(0, ng // 4, gg_body, 0)
            for p in range(3):
                gwait(ng - 1, p, sems[p])
            pltpu.sync_copy(
                out_v,
                out_hbm.at[h, pl.ds(b * N + nbase + ci * CQ, CQ)])
            return 0

        lax.fori_loop(0, nchunks, chunk_body, 0)

    return k(v_flat, idx4, w_flat)


def kernel(query, query_pos, W_so, b_so, W_aw, b_aw, W_vp, b_vp, W_op, b_op,
           W_out, b_out):
    wvp = W_vp.reshape(NH, DH, C)
    bvp = b_vp.reshape(NH, DH)
    wso = W_so.reshape(NH, NP, 2, C)
    bso = b_so.reshape(NH, NP, 2)
    waw = W_aw.reshape(NH, NP, C)
    baw = b_aw.reshape(NH, NP)

    v, idx, w = pl.pallas_call(
        _ph1_body,
        grid=(B, N // BN),
        in_specs=[
            pl.BlockSpec((1, BN, C), lambda b, i: (b, i, 0)),
            pl.BlockSpec((1, BN, C), lambda b, i: (b, i, 0)),
            pl.BlockSpec((NH, DH, C), lambda b, i: (0, 0, 0)),
            pl.BlockSpec((NH, DH), lambda b, i: (0, 0)),
            pl.BlockSpec((NH, NP, C), lambda b, i: (0, 0, 0)),
            pl.BlockSpec((NH, NP), lambda b, i: (0, 0)),
            pl.BlockSpec((NH, NP, C), lambda b, i: (0, 0, 0)),
            pl.BlockSpec((NH, NP), lambda b, i: (0, 0)),
            pl.BlockSpec((NH, NP, C), lambda b, i: (0, 0, 0)),
            pl.BlockSpec((NH, NP), lambda b, i: (0, 0)),
        ],
        out_specs=[
            pl.BlockSpec((1, NH, BN, 2 * DH), lambda b, i: (b, 0, i, 0)),
            pl.BlockSpec((1, NH, BN, KI), lambda b, i: (b, 0, i, 0)),
            pl.BlockSpec((1, NH, BN, KW), lambda b, i: (b, 0, i, 0)),
        ],
        out_shape=[
            jax.ShapeDtypeStruct((B, NH, N, 2 * DH), jnp.float32),
            jax.ShapeDtypeStruct((B, NH, N, KI), jnp.int32),
            jax.ShapeDtypeStruct((B, NH, N, KW), jnp.float32),
        ],
        compiler_params=pltpu.CompilerParams(
            dimension_semantics=("parallel", "parallel")),
    )(query, query_pos, wvp, bvp, wso[:, :, 0], bso[:, :, 0], wso[:, :, 1],
      bso[:, :, 1], waw, baw)

    out_s = _sc_gather(v.reshape(ITEMS, 2 * DH), idx.reshape(ITEMS, KI),
                       w.reshape(ITEMS, KW))

    qfl = query.reshape(B * N, C)
    wop_h = jnp.transpose(W_op.reshape(C, NH, DH), (1, 0, 2))
    BN3 = 1000
    out = pl.pallas_call(
        _ph3_body,
        grid=(B * N // BN3,),
        in_specs=[
            pl.BlockSpec((BN3, DH), lambda i: (i, 0)),
            pl.BlockSpec((BN3, DH), lambda i: (i, 0)),
            pl.BlockSpec((BN3, DH), lambda i: (i, 0)),
            pl.BlockSpec((BN3, DH), lambda i: (i, 0)),
            pl.BlockSpec((BN3, C), lambda i: (i, 0)),
            pl.BlockSpec((NH, C, DH), lambda i: (0, 0, 0)),
            pl.BlockSpec((C,), lambda i: (0,)),
            pl.BlockSpec((C, C), lambda i: (0, 0)),
            pl.BlockSpec((C,), lambda i: (0,)),
        ],
        out_specs=pl.BlockSpec((BN3, C), lambda i: (i, 0)),
        out_shape=jax.ShapeDtypeStruct((B * N, C), jnp.float32),
        compiler_params=pltpu.CompilerParams(
            dimension_semantics=("parallel",)),
    )(out_s[0], out_s[1], out_s[2], out_s[3], qfl, wop_h, b_op, W_out, b_out)
    return out.reshape(B, N, C)
